# Initial kernel scaffold; baseline (speedup 1.0000x reference)
import jax
import jax.numpy as jnp
from jax.experimental import pallas as pl


def kernel(x, edge_index, W):
    raise NotImplementedError("write your pallas kernel here")


# speedup vs baseline: 18.2716x; 18.2716x over previous
import functools

import jax
import jax.numpy as jnp
from jax import lax
from jax.experimental import pallas as pl
from jax.experimental.pallas import tpu as pltpu
from jax.experimental.pallas import tpu_sc as plsc

N = 10000
E = 320000
D = 128

NC = 2
NS = 16
NW = NC * NS

NBLK = 79
NPAD = NBLK * 128
EPT = NPAD
EPAD = NW * EPT
CHUNK = 128
NCHUNK = EPT // CHUNK
RPT = NPAD // NS

_MESH = plsc.VectorSubcoreMesh(core_axis_name="c", subcore_axis_name="s")


@functools.partial(
    pl.kernel,
    out_type=jax.ShapeDtypeStruct((NC * NPAD,), jnp.float32),
    mesh=_MESH,
    scratch_types=[
        pltpu.VMEM_SHARED((NPAD,), jnp.float32),
        pltpu.VMEM((CHUNK,), jnp.int32),
        pltpu.VMEM((CHUNK,), jnp.float32),
        pltpu.VMEM((RPT,), jnp.float32),
    ],
)
def _sc_degree(dst_hbm, zeros_hbm, degp_hbm, acc, idxv, onesv, stg):
    cid = lax.axis_index("c")
    sid = lax.axis_index("s")
    wid = cid * NS + sid

    for j in range(CHUNK // 16):
        onesv[pl.ds(16 * j, 16)] = jnp.ones((16,), jnp.float32)
    pltpu.sync_copy(zeros_hbm, stg)
    pltpu.sync_copy(stg, acc.at[pl.ds(sid * RPT, RPT)])
    plsc.subcore_barrier()

    def body(g, carry):
        base = wid * EPT + g * CHUNK
        pltpu.sync_copy(dst_hbm.at[pl.ds(base, CHUNK)], idxv)
        pltpu.sync_copy(onesv, acc.at[idxv], add=True)
        return carry

    lax.fori_loop(0, NCHUNK, body, 0)
    plsc.subcore_barrier()

    pltpu.sync_copy(acc.at[pl.ds(sid * RPT, RPT)], stg)
    pltpu.sync_copy(stg, degp_hbm.at[pl.ds(cid * NPAD + sid * RPT, RPT)])


@functools.partial(
    pl.kernel,
    out_type=jax.ShapeDtypeStruct((NC * NPAD, D), jnp.float32),
    mesh=_MESH,
    scratch_types=[
        pltpu.VMEM_SHARED((NPAD, D), jnp.float32),
        pltpu.VMEM((CHUNK,), jnp.int32),
        pltpu.VMEM((CHUNK,), jnp.int32),
        pltpu.VMEM((CHUNK, D), jnp.float32),
        pltpu.SemaphoreType.DMA,
    ],
)
def _sc_aggregate(src_hbm, dst_hbm, h2_hbm, zrows_hbm, accp_hbm,
                  acc, sidx, didx, rows, sem):
    cid = lax.axis_index("c")
    sid = lax.axis_index("s")
    wid = cid * NS + sid

    pltpu.sync_copy(zrows_hbm, acc.at[pl.ds(sid * RPT, RPT)])
    plsc.subcore_barrier()

    def body(g, carry):
        base = wid * EPT + g * CHUNK
        pltpu.sync_copy(src_hbm.at[pl.ds(base, CHUNK)], sidx)
        pltpu.sync_copy(dst_hbm.at[pl.ds(base, CHUNK)], didx)
        pltpu.async_copy(h2_hbm.at[sidx], rows, sem).wait()
        pltpu.sync_copy(rows, acc.at[didx], add=True)
        return carry

    lax.fori_loop(0, NCHUNK, body, 0)
    plsc.subcore_barrier()

    pltpu.sync_copy(
        acc.at[pl.ds(sid * RPT, RPT)],
        accp_hbm.at[pl.ds(cid * NPAD + sid * RPT, RPT)],
    )


def _dinv_diag(degp_ref, i):
    deg = degp_ref[pl.ds(i, 1), :] + degp_ref[pl.ds(NBLK + i, 1), :] + 1.0
    dinv = lax.rsqrt(deg)
    r = lax.broadcasted_iota(jnp.int32, (128, 128), 0)
    c = lax.broadcasted_iota(jnp.int32, (128, 128), 1)
    return jnp.where(r == c, dinv, 0.0)


def _tc_prescale_body(x_ref, w_ref, degp_ref, h2_ref):
    i = pl.program_id(0)
    diag = _dinv_diag(degp_ref, i)
    h = jnp.dot(x_ref[...], w_ref[...], preferred_element_type=jnp.float32)
    h2_ref[...] = jnp.dot(diag, h, preferred_element_type=jnp.float32)


def _tc_combine_body(accp_ref, h2_ref, degp_ref, out_ref):
    i = pl.program_id(0)
    diag = _dinv_diag(degp_ref, i)
    s = accp_ref[0] + accp_ref[1] + h2_ref[...]
    out_ref[...] = jnp.dot(diag, s, preferred_element_type=jnp.float32)


_tc_prescale = pl.pallas_call(
    _tc_prescale_body,
    grid=(NBLK,),
    in_specs=[
        pl.BlockSpec((128, D), lambda i: (i, 0)),
        pl.BlockSpec((D, D), lambda i: (0, 0)),
        pl.BlockSpec((2 * NBLK, 128), lambda i: (0, 0)),
    ],
    out_specs=pl.BlockSpec((128, D), lambda i: (i, 0)),
    out_shape=jax.ShapeDtypeStruct((NPAD, D), jnp.float32),
)

_tc_combine = pl.pallas_call(
    _tc_combine_body,
    grid=(NBLK,),
    in_specs=[
        pl.BlockSpec((NC, 128, D), lambda i: (0, i, 0)),
        pl.BlockSpec((128, D), lambda i: (i, 0)),
        pl.BlockSpec((2 * NBLK, 128), lambda i: (0, 0)),
    ],
    out_specs=pl.BlockSpec((128, D), lambda i: (i, 0)),
    out_shape=jax.ShapeDtypeStruct((NPAD, D), jnp.float32),
)


def kernel(x, edge_index, W):
    src = edge_index[0].astype(jnp.int32)
    dst = edge_index[1].astype(jnp.int32)
    npadrows = NPAD - N
    pad = N + (lax.iota(jnp.int32, EPAD - E) % npadrows)
    srcp = jnp.concatenate([src, pad])
    dstp = jnp.concatenate([dst, pad])
    xp = jnp.zeros((NPAD, D), jnp.float32).at[:N].set(x)

    za = jnp.zeros((RPT,), jnp.float32)
    zc = jnp.zeros((RPT, D), jnp.float32)

    degp = _sc_degree(dstp, za).reshape(2 * NBLK, 128)
    h2 = _tc_prescale(xp, W, degp)
    accp = _sc_aggregate(srcp, dstp, h2, zc).reshape(NC, NPAD, D)
    out = _tc_combine(accp, h2, degp)
    return out[:N]


# speedup vs baseline: 27.2185x; 1.4897x over previous
import functools

import jax
import jax.numpy as jnp
from jax import lax
from jax.experimental import pallas as pl
from jax.experimental.pallas import tpu as pltpu
from jax.experimental.pallas import tpu_sc as plsc

N = 10000
E = 320000
D = 128

NC = 2
NS = 16
NW = NC * NS

NBLK = 79
NPAD = NBLK * 128
EPT = NPAD
EPAD = NW * EPT
CHUNK = 128
NCHUNK = EPT // CHUNK
RPT = NPAD // NS

_MESH = plsc.VectorSubcoreMesh(core_axis_name="c", subcore_axis_name="s")


@functools.partial(
    pl.kernel,
    out_type=jax.ShapeDtypeStruct((NC * NPAD,), jnp.float32),
    mesh=_MESH,
    scratch_types=[
        pltpu.VMEM_SHARED((NPAD,), jnp.float32),
        pltpu.VMEM((2, 2, CHUNK), jnp.int32),
        pltpu.VMEM((CHUNK,), jnp.float32),
        pltpu.VMEM((RPT,), jnp.float32),
        pltpu.SemaphoreType.DMA,
        pltpu.SemaphoreType.DMA,
    ],
)
def _sc_degree(eidx_hbm, zeros_hbm, degp_hbm, acc, eidx, onesv, stg, s0, s1):
    cid = lax.axis_index("c")
    sid = lax.axis_index("s")
    wid = cid * NS + sid
    qbase = wid * NCHUNK

    for j in range(CHUNK // 16):
        onesv[pl.ds(16 * j, 16)] = jnp.ones((16,), jnp.float32)
    pltpu.sync_copy(zeros_hbm, stg)
    pltpu.sync_copy(stg, acc.at[pl.ds(sid * RPT, RPT)])
    plsc.subcore_barrier()

    sems = (s0, s1)

    def load(g, b):
        pltpu.async_copy(eidx_hbm.at[qbase + g], eidx.at[b], sems[b])

    def drain(g, b):
        pltpu.make_async_copy(eidx_hbm.at[qbase + g], eidx.at[b], sems[b]).wait()
        pltpu.sync_copy(onesv, acc.at[eidx.at[b].at[1]], add=True)

    load(0, 0)

    def body(i, carry):
        g = i * 2
        load(g + 1, 1)
        drain(g, 0)

        @pl.when(g + 2 < NCHUNK)
        def _():
            load(g + 2, 0)

        drain(g + 1, 1)
        return carry

    lax.fori_loop(0, NCHUNK // 2, body, 0)
    drain(NCHUNK - 1, 0)
    plsc.subcore_barrier()

    pltpu.sync_copy(acc.at[pl.ds(sid * RPT, RPT)], stg)
    pltpu.sync_copy(stg, degp_hbm.at[pl.ds(cid * NPAD + sid * RPT, RPT)])


@functools.partial(
    pl.kernel,
    out_type=jax.ShapeDtypeStruct((NC * NPAD, D), jnp.float32),
    mesh=_MESH,
    scratch_types=[
        pltpu.VMEM_SHARED((NPAD, D), jnp.float32),
        pltpu.VMEM((2, 2, CHUNK), jnp.int32),
        pltpu.VMEM((2, CHUNK, D), jnp.float32),
        pltpu.SemaphoreType.DMA,
        pltpu.SemaphoreType.DMA,
        pltpu.SemaphoreType.DMA,
        pltpu.SemaphoreType.DMA,
    ],
)
def _sc_aggregate(eidx_hbm, h2_hbm, zrows_hbm, accp_hbm,
                  acc, eidx, rows, si0, si1, sr0, sr1):
    cid = lax.axis_index("c")
    sid = lax.axis_index("s")
    wid = cid * NS + sid
    qbase = wid * NCHUNK

    pltpu.sync_copy(zrows_hbm, acc.at[pl.ds(sid * RPT, RPT)])
    plsc.subcore_barrier()

    isems = (si0, si1)
    rsems = (sr0, sr1)

    def start(g, b):
        pltpu.make_async_copy(eidx_hbm.at[qbase + g], eidx.at[b], isems[b]).wait()
        pltpu.async_copy(h2_hbm.at[eidx.at[b].at[0]], rows.at[b], rsems[b])

    def load_idx(g, b):
        pltpu.async_copy(eidx_hbm.at[qbase + g], eidx.at[b], isems[b])

    def drain(g, b):
        pltpu.make_async_copy(h2_hbm.at[eidx.at[b].at[0]], rows.at[b],
                              rsems[b]).wait()
        pltpu.sync_copy(rows.at[b], acc.at[eidx.at[b].at[1]], add=True)

    load_idx(0, 0)
    start(0, 0)

    def body(i, carry):
        g = i * 2
        load_idx(g + 1, 1)
        start(g + 1, 1)
        drain(g, 0)

        @pl.when(g + 2 < NCHUNK)
        def _():
            load_idx(g + 2, 0)
            start(g + 2, 0)

        drain(g + 1, 1)
        return carry

    lax.fori_loop(0, NCHUNK // 2, body, 0)
    drain(NCHUNK - 1, 0)
    plsc.subcore_barrier()

    pltpu.sync_copy(
        acc.at[pl.ds(sid * RPT, RPT)],
        accp_hbm.at[pl.ds(cid * NPAD + sid * RPT, RPT)],
    )


def _dinv_diag(degp_ref, i):
    deg = degp_ref[pl.ds(i, 1), :] + degp_ref[pl.ds(NBLK + i, 1), :] + 1.0
    dinv = lax.rsqrt(deg)
    r = lax.broadcasted_iota(jnp.int32, (128, 128), 0)
    c = lax.broadcasted_iota(jnp.int32, (128, 128), 1)
    return jnp.where(r == c, dinv, 0.0)


def _tc_prescale_body(x_ref, w_ref, degp_ref, h2_ref):
    i = pl.program_id(0)
    diag = _dinv_diag(degp_ref, i)
    h = jnp.dot(x_ref[...], w_ref[...], preferred_element_type=jnp.float32)
    h2_ref[...] = jnp.dot(diag, h, preferred_element_type=jnp.float32)


def _tc_combine_body(accp_ref, h2_ref, degp_ref, out_ref):
    i = pl.program_id(0)
    diag = _dinv_diag(degp_ref, i)
    s = accp_ref[0] + accp_ref[1] + h2_ref[...]
    out_ref[...] = jnp.dot(diag, s, preferred_element_type=jnp.float32)


_tc_prescale = pl.pallas_call(
    _tc_prescale_body,
    grid=(NBLK,),
    in_specs=[
        pl.BlockSpec((128, D), lambda i: (i, 0)),
        pl.BlockSpec((D, D), lambda i: (0, 0)),
        pl.BlockSpec((2 * NBLK, 128), lambda i: (0, 0)),
    ],
    out_specs=pl.BlockSpec((128, D), lambda i: (i, 0)),
    out_shape=jax.ShapeDtypeStruct((NPAD, D), jnp.float32),
)

_tc_combine = pl.pallas_call(
    _tc_combine_body,
    grid=(NBLK,),
    in_specs=[
        pl.BlockSpec((NC, 128, D), lambda i: (0, i, 0)),
        pl.BlockSpec((128, D), lambda i: (i, 0)),
        pl.BlockSpec((2 * NBLK, 128), lambda i: (0, 0)),
    ],
    out_specs=pl.BlockSpec((128, D), lambda i: (i, 0)),
    out_shape=jax.ShapeDtypeStruct((NPAD, D), jnp.float32),
)


def kernel(x, edge_index, W):
    src = edge_index[0].astype(jnp.int32)
    dst = edge_index[1].astype(jnp.int32)
    npadrows = NPAD - N
    pad = N + (lax.iota(jnp.int32, EPAD - E) % npadrows)
    srcp = jnp.concatenate([src, pad])
    dstp = jnp.concatenate([dst, pad])
    eidx = jnp.stack(
        [srcp.reshape(NW * NCHUNK, CHUNK), dstp.reshape(NW * NCHUNK, CHUNK)],
        axis=1,
    )
    xp = jnp.zeros((NPAD, D), jnp.float32).at[:N].set(x)

    za = jnp.zeros((RPT,), jnp.float32)
    zc = jnp.zeros((RPT, D), jnp.float32)

    degp = _sc_degree(eidx, za).reshape(2 * NBLK, 128)
    h2 = _tc_prescale(xp, W, degp)
    accp = _sc_aggregate(eidx, h2, zc).reshape(NC, NPAD, D)
    out = _tc_combine(accp, h2, degp)
    return out[:N]


# speedup vs baseline: 27.9016x; 1.0251x over previous
import functools

import jax
import jax.numpy as jnp
from jax import lax
from jax.experimental import pallas as pl
from jax.experimental.pallas import tpu as pltpu
from jax.experimental.pallas import tpu_sc as plsc

N = 10000
E = 320000
D = 128

NC = 2
NS = 16
NW = NC * NS

NBLK = 79
NPAD = NBLK * 128
CHUNK = 80
NCHUNK = 128
EPT = NCHUNK * CHUNK
EPAD = NW * EPT
RPT = NPAD // NS
NBUF = 4
NIDX = 8

_MESH = plsc.VectorSubcoreMesh(core_axis_name="c", subcore_axis_name="s")


@functools.partial(
    pl.kernel,
    out_type=jax.ShapeDtypeStruct((NC * NPAD,), jnp.float32),
    mesh=_MESH,
    scratch_types=[
        pltpu.VMEM_SHARED((NPAD,), jnp.float32),
        pltpu.VMEM((NCHUNK, 2, CHUNK), jnp.int32),
        pltpu.VMEM((CHUNK,), jnp.float32),
        pltpu.VMEM((RPT,), jnp.float32),
        pltpu.SemaphoreType.DMA,
        pltpu.SemaphoreType.DMA,
    ],
)
def _sc_degree(eidx_hbm, zeros_hbm, degp_hbm, acc, eidx, onesv, stg, s0, s1):
    cid = lax.axis_index("c")
    sid = lax.axis_index("s")
    wid = cid * NS + sid

    pltpu.sync_copy(eidx_hbm.at[pl.ds(wid * NCHUNK, NCHUNK)], eidx)
    for j in range(CHUNK // 16):
        onesv[pl.ds(16 * j, 16)] = jnp.ones((16,), jnp.float32)
    pltpu.sync_copy(zeros_hbm, stg)
    pltpu.sync_copy(stg, acc.at[pl.ds(sid * RPT, RPT)])
    plsc.subcore_barrier()

    sems = (s0, s1)

    def fire(g, b):
        pltpu.async_copy(onesv, acc.at[eidx.at[g].at[1]], sems[b], add=True)

    def wait(g, b):
        pltpu.make_async_copy(onesv, acc.at[eidx.at[g].at[1]], sems[b]).wait()

    fire(0, 0)

    def body(i, carry):
        g = i * 2
        fire(g + 1, 1)
        wait(g, 0)

        @pl.when(g + 2 < NCHUNK)
        def _():
            fire(g + 2, 0)

        wait(g + 1, 1)
        return carry

    lax.fori_loop(0, NCHUNK // 2, body, 0)
    plsc.subcore_barrier()

    pltpu.sync_copy(acc.at[pl.ds(sid * RPT, RPT)], stg)
    pltpu.sync_copy(stg, degp_hbm.at[pl.ds(cid * NPAD + sid * RPT, RPT)])


@functools.partial(
    pl.kernel,
    out_type=jax.ShapeDtypeStruct((NC * NPAD, D), jnp.float32),
    mesh=_MESH,
    scratch_types=[
        pltpu.VMEM_SHARED((NPAD, D), jnp.float32),
        pltpu.VMEM((NIDX, 2, CHUNK), jnp.int32),
        pltpu.VMEM((NBUF, CHUNK, D), jnp.float32),
        pltpu.SemaphoreType.DMA,
        pltpu.SemaphoreType.DMA,
        pltpu.SemaphoreType.DMA,
        pltpu.SemaphoreType.DMA,
        pltpu.SemaphoreType.DMA,
        pltpu.SemaphoreType.DMA,
        pltpu.SemaphoreType.DMA,
        pltpu.SemaphoreType.DMA,
        pltpu.SemaphoreType.DMA,
        pltpu.SemaphoreType.DMA,
        pltpu.SemaphoreType.DMA,
        pltpu.SemaphoreType.DMA,
    ],
)
def _sc_aggregate(eidx_hbm, h2_hbm, zrows_hbm, accp_hbm, acc, eidx, rows,
                  i0, i1, i2, i3, i4, i5, i6, i7, r0, r1, r2, r3):
    cid = lax.axis_index("c")
    sid = lax.axis_index("s")
    wid = cid * NS + sid
    qbase = wid * NCHUNK

    pltpu.sync_copy(zrows_hbm, acc.at[pl.ds(sid * RPT, RPT)])
    plsc.subcore_barrier()

    isems = (i0, i1, i2, i3, i4, i5, i6, i7)
    rsems = (r0, r1, r2, r3)

    def fire_idx(g, s):
        pltpu.async_copy(eidx_hbm.at[qbase + g], eidx.at[s], isems[s])

    def fire_gather(g, s, b):
        pltpu.make_async_copy(eidx_hbm.at[qbase + g], eidx.at[s],
                              isems[s]).wait()
        pltpu.async_copy(h2_hbm.at[eidx.at[s].at[0]], rows.at[b], rsems[b])

    def drain(g, s, b):
        pltpu.make_async_copy(h2_hbm.at[eidx.at[s].at[0]], rows.at[b],
                              rsems[b]).wait()
        pltpu.sync_copy(rows.at[b], acc.at[eidx.at[s].at[1]], add=True)

    for j in range(NBUF + 2):
        fire_idx(j, j)
    for b in range(NBUF - 1):
        fire_gather(b, b, b)

    def body(i, carry):
        for u in range(NIDX):
            g = i * NIDX + u

            @pl.when(g + NBUF + 2 < NCHUNK)
            def _():
                fire_idx(g + NBUF + 2, (u + NBUF + 2) % NIDX)

            @pl.when(g + NBUF - 1 < NCHUNK)
            def _():
                fire_gather(g + NBUF - 1, (u + NBUF - 1) % NIDX,
                            (u + NBUF - 1) % NBUF)

            drain(g, u % NIDX, u % NBUF)
        return carry

    lax.fori_loop(0, NCHUNK // NIDX, body, 0)
    plsc.subcore_barrier()

    pltpu.sync_copy(
        acc.at[pl.ds(sid * RPT, RPT)],
        accp_hbm.at[pl.ds(cid * NPAD + sid * RPT, RPT)],
    )


def _dinv_diag(degp_ref, i):
    deg = degp_ref[pl.ds(i, 1), :] + degp_ref[pl.ds(NBLK + i, 1), :] + 1.0
    dinv = lax.rsqrt(deg)
    r = lax.broadcasted_iota(jnp.int32, (128, 128), 0)
    c = lax.broadcasted_iota(jnp.int32, (128, 128), 1)
    return jnp.where(r == c, dinv, 0.0)


def _tc_matmul_body(x_ref, w_ref, h_ref):
    h_ref[...] = jnp.dot(x_ref[...], w_ref[...],
                         preferred_element_type=jnp.float32)


def _tc_prescale_body(h_ref, degp_ref, h2_ref):
    i = pl.program_id(0)
    h2_ref[...] = jnp.dot(_dinv_diag(degp_ref, i), h_ref[...],
                          preferred_element_type=jnp.float32)


def _tc_combine_body(accp_ref, h2_ref, degp_ref, out_ref):
    i = pl.program_id(0)
    s = accp_ref[0] + accp_ref[1] + h2_ref[...]
    out_ref[...] = jnp.dot(_dinv_diag(degp_ref, i), s,
                           preferred_element_type=jnp.float32)


_tc_matmul = pl.pallas_call(
    _tc_matmul_body,
    grid=(NBLK,),
    in_specs=[
        pl.BlockSpec((128, D), lambda i: (i, 0)),
        pl.BlockSpec((D, D), lambda i: (0, 0)),
    ],
    out_specs=pl.BlockSpec((128, D), lambda i: (i, 0)),
    out_shape=jax.ShapeDtypeStruct((NPAD, D), jnp.float32),
)

_tc_prescale = pl.pallas_call(
    _tc_prescale_body,
    grid=(NBLK,),
    in_specs=[
        pl.BlockSpec((128, D), lambda i: (i, 0)),
        pl.BlockSpec((2 * NBLK, 128), lambda i: (0, 0)),
    ],
    out_specs=pl.BlockSpec((128, D), lambda i: (i, 0)),
    out_shape=jax.ShapeDtypeStruct((NPAD, D), jnp.float32),
)

_tc_combine = pl.pallas_call(
    _tc_combine_body,
    grid=(NBLK,),
    in_specs=[
        pl.BlockSpec((NC, 128, D), lambda i: (0, i, 0)),
        pl.BlockSpec((128, D), lambda i: (i, 0)),
        pl.BlockSpec((2 * NBLK, 128), lambda i: (0, 0)),
    ],
    out_specs=pl.BlockSpec((128, D), lambda i: (i, 0)),
    out_shape=jax.ShapeDtypeStruct((NPAD, D), jnp.float32),
)


def kernel(x, edge_index, W):
    src = edge_index[0].astype(jnp.int32)
    dst = edge_index[1].astype(jnp.int32)
    npadrows = NPAD - N
    pad = N + (lax.iota(jnp.int32, EPAD - E) % npadrows)
    srcp = jnp.concatenate([src, pad])
    dstp = jnp.concatenate([dst, pad])
    eidx = jnp.stack(
        [srcp.reshape(NW * NCHUNK, CHUNK), dstp.reshape(NW * NCHUNK, CHUNK)],
        axis=1,
    )
    xp = jnp.zeros((NPAD, D), jnp.float32).at[:N].set(x)

    za = jnp.zeros((RPT,), jnp.float32)
    zc = jnp.zeros((RPT, D), jnp.float32)

    degp = _sc_degree(eidx, za).reshape(2 * NBLK, 128)
    h = _tc_matmul(xp, W)
    h2 = _tc_prescale(h, degp)
    accp = _sc_aggregate(eidx, h2, zc).reshape(NC, NPAD, D)
    out = _tc_combine(accp, h2, degp)
    return out[:N]


# speedup vs baseline: 32.2730x; 1.1567x over previous
import functools

import jax
import jax.numpy as jnp
from jax import lax
from jax.experimental import pallas as pl
from jax.experimental.pallas import tpu as pltpu
from jax.experimental.pallas import tpu_sc as plsc

N = 10000
E = 320000
D = 128

NC = 2
NS = 16
NW = NC * NS

NBLK = 79
NPAD = NBLK * 128
CHUNK = 80
NCHUNK = 128
EPT = NCHUNK * CHUNK
EPAD = NW * EPT
RPT = NPAD // NS
NBUF = 4
NIDX = 8

_MESH = plsc.VectorSubcoreMesh(core_axis_name="c", subcore_axis_name="s")


@functools.partial(
    pl.kernel,
    out_type=jax.ShapeDtypeStruct((NC * NPAD,), jnp.float32),
    mesh=_MESH,
    scratch_types=[
        pltpu.VMEM_SHARED((NPAD,), jnp.float32),
        pltpu.VMEM((NCHUNK, 2, CHUNK), jnp.int32),
        pltpu.VMEM((CHUNK,), jnp.float32),
        pltpu.VMEM((RPT,), jnp.float32),
        pltpu.SemaphoreType.DMA,
        pltpu.SemaphoreType.DMA,
    ],
)
def _sc_degree(eidx_hbm, zeros_hbm, degp_hbm, acc, eidx, onesv, stg, s0, s1):
    cid = lax.axis_index("c")
    sid = lax.axis_index("s")
    wid = cid * NS + sid

    pltpu.sync_copy(eidx_hbm.at[pl.ds(wid * NCHUNK, NCHUNK)], eidx)
    for j in range(CHUNK // 16):
        onesv[pl.ds(16 * j, 16)] = jnp.ones((16,), jnp.float32)
    pltpu.sync_copy(zeros_hbm, stg)
    pltpu.sync_copy(stg, acc.at[pl.ds(sid * RPT, RPT)])
    plsc.subcore_barrier()

    sems = (s0, s1)

    def fire(g, b):
        pltpu.async_copy(onesv, acc.at[eidx.at[g].at[1]], sems[b], add=True)

    def wait(g, b):
        pltpu.make_async_copy(onesv, acc.at[eidx.at[g].at[1]], sems[b]).wait()

    fire(0, 0)

    def body(i, carry):
        g = i * 2
        fire(g + 1, 1)
        wait(g, 0)

        @pl.when(g + 2 < NCHUNK)
        def _():
            fire(g + 2, 0)

        wait(g + 1, 1)
        return carry

    lax.fori_loop(0, NCHUNK // 2, body, 0)
    plsc.subcore_barrier()

    pltpu.sync_copy(acc.at[pl.ds(sid * RPT, RPT)], stg)
    pltpu.sync_copy(stg, degp_hbm.at[pl.ds(cid * NPAD + sid * RPT, RPT)])


@functools.partial(
    pl.kernel,
    out_type=jax.ShapeDtypeStruct((NC * NPAD, D), jnp.float32),
    mesh=_MESH,
    scratch_types=[
        pltpu.VMEM_SHARED((NPAD, D), jnp.float32),
        pltpu.VMEM((NIDX, 2, CHUNK), jnp.int32),
        pltpu.VMEM((NBUF, CHUNK, D), jnp.float32),
        pltpu.SemaphoreType.DMA,
        pltpu.SemaphoreType.DMA,
        pltpu.SemaphoreType.DMA,
        pltpu.SemaphoreType.DMA,
        pltpu.SemaphoreType.DMA,
        pltpu.SemaphoreType.DMA,
        pltpu.SemaphoreType.DMA,
        pltpu.SemaphoreType.DMA,
        pltpu.SemaphoreType.DMA,
        pltpu.SemaphoreType.DMA,
        pltpu.SemaphoreType.DMA,
        pltpu.SemaphoreType.DMA,
    ],
)
def _sc_aggregate(eidx_hbm, h2_hbm, zrows_hbm, accp_hbm, acc, eidx, rows,
                  i0, i1, i2, i3, i4, i5, i6, i7, r0, r1, r2, r3):
    cid = lax.axis_index("c")
    sid = lax.axis_index("s")
    wid = cid * NS + sid
    qbase = wid * NCHUNK

    pltpu.sync_copy(zrows_hbm, acc.at[pl.ds(sid * RPT, RPT)])
    plsc.subcore_barrier()

    isems = (i0, i1, i2, i3, i4, i5, i6, i7)
    rsems = (r0, r1, r2, r3)

    def fire_idx(g, s):
        pltpu.async_copy(eidx_hbm.at[qbase + g], eidx.at[s], isems[s])

    def fire_gather(g, s, b):
        pltpu.make_async_copy(eidx_hbm.at[qbase + g], eidx.at[s],
                              isems[s]).wait()
        pltpu.async_copy(h2_hbm.at[eidx.at[s].at[0]], rows.at[b], rsems[b])

    def drain(g, s, b):
        pltpu.make_async_copy(h2_hbm.at[eidx.at[s].at[0]], rows.at[b],
                              rsems[b]).wait()
        pltpu.sync_copy(rows.at[b], acc.at[eidx.at[s].at[1]], add=True)

    for j in range(NBUF + 2):
        fire_idx(j, j)
    for b in range(NBUF - 1):
        fire_gather(b, b, b)

    def body(i, carry):
        for u in range(NIDX):
            g = i * NIDX + u

            @pl.when(g + NBUF + 2 < NCHUNK)
            def _():
                fire_idx(g + NBUF + 2, (u + NBUF + 2) % NIDX)

            @pl.when(g + NBUF - 1 < NCHUNK)
            def _():
                fire_gather(g + NBUF - 1, (u + NBUF - 1) % NIDX,
                            (u + NBUF - 1) % NBUF)

            drain(g, u % NIDX, u % NBUF)
        return carry

    lax.fori_loop(0, NCHUNK // NIDX, body, 0)
    plsc.subcore_barrier()

    pltpu.sync_copy(
        acc.at[pl.ds(sid * RPT, RPT)],
        accp_hbm.at[pl.ds(cid * NPAD + sid * RPT, RPT)],
    )


def _dinv_diag(degp_ref, i):
    deg = degp_ref[pl.ds(i, 1), :] + degp_ref[pl.ds(NBLK + i, 1), :] + 1.0
    dinv = lax.rsqrt(deg)
    r = lax.broadcasted_iota(jnp.int32, (128, 128), 0)
    c = lax.broadcasted_iota(jnp.int32, (128, 128), 1)
    return jnp.where(r == c, dinv, 0.0)


def _tc_prescale_body(x_ref, w_ref, degp_ref, h2_ref):
    i = pl.program_id(0)
    h = jnp.dot(x_ref[...], w_ref[...], preferred_element_type=jnp.float32)
    h2_ref[...] = jnp.dot(_dinv_diag(degp_ref, i), h,
                          preferred_element_type=jnp.float32)


def _tc_combine_body(accp_ref, h2_ref, degp_ref, out_ref):
    i = pl.program_id(0)
    s = accp_ref[0] + accp_ref[1] + h2_ref[...]
    out_ref[...] = jnp.dot(_dinv_diag(degp_ref, i), s,
                           preferred_element_type=jnp.float32)


_tc_prescale = pl.pallas_call(
    _tc_prescale_body,
    grid=(NBLK,),
    in_specs=[
        pl.BlockSpec((128, D), lambda i: (i, 0)),
        pl.BlockSpec((D, D), lambda i: (0, 0)),
        pl.BlockSpec((2 * NBLK, 128), lambda i: (0, 0)),
    ],
    out_specs=pl.BlockSpec((128, D), lambda i: (i, 0)),
    out_shape=jax.ShapeDtypeStruct((NPAD, D), jnp.float32),
)

_tc_combine = pl.pallas_call(
    _tc_combine_body,
    grid=(NBLK,),
    in_specs=[
        pl.BlockSpec((NC, 128, D), lambda i: (0, i, 0)),
        pl.BlockSpec((128, D), lambda i: (i, 0)),
        pl.BlockSpec((2 * NBLK, 128), lambda i: (0, 0)),
    ],
    out_specs=pl.BlockSpec((128, D), lambda i: (i, 0)),
    out_shape=jax.ShapeDtypeStruct((N, D), jnp.float32),
)


def kernel(x, edge_index, W):
    src = edge_index[0].astype(jnp.int32)
    dst = edge_index[1].astype(jnp.int32)
    npadrows = NPAD - N
    pad = N + (lax.iota(jnp.int32, EPAD - E) % npadrows)
    srcp = jnp.concatenate([src, pad])
    dstp = jnp.concatenate([dst, pad])
    eidx = jnp.stack(
        [srcp.reshape(NW * NCHUNK, CHUNK), dstp.reshape(NW * NCHUNK, CHUNK)],
        axis=1,
    )
    xp = jnp.zeros((NPAD, D), jnp.float32).at[:N].set(x)

    za = jnp.zeros((RPT,), jnp.float32)
    zc = jnp.zeros((RPT, D), jnp.float32)

    degp = _sc_degree(eidx, za).reshape(2 * NBLK, 128)
    h2 = _tc_prescale(xp, W, degp)
    accp = _sc_aggregate(eidx, h2, zc).reshape(NC, NPAD, D)
    return _tc_combine(accp, h2, degp)


# speedup vs baseline: 32.8746x; 1.0186x over previous
import functools

import jax
import jax.numpy as jnp
from jax import lax
from jax.experimental import pallas as pl
from jax.experimental.pallas import tpu as pltpu
from jax.experimental.pallas import tpu_sc as plsc

N = 10000
E = 320000
D = 128

NC = 2
NS = 16
NW = NC * NS

NBLK = 79
NPAD = NBLK * 128
CHUNK = 80
NCHUNK = 128
EPT = NCHUNK * CHUNK
EPAD = NW * EPT
RPT = NPAD // NS
NBUF = 4
NIDX = 8

_MESH = plsc.VectorSubcoreMesh(core_axis_name="c", subcore_axis_name="s")


@functools.partial(
    pl.kernel,
    out_type=jax.ShapeDtypeStruct((NC * NPAD,), jnp.float32),
    mesh=_MESH,
    scratch_types=[
        pltpu.VMEM_SHARED((NPAD,), jnp.float32),
        pltpu.VMEM((NCHUNK, 2, CHUNK), jnp.int32),
        pltpu.VMEM((CHUNK,), jnp.float32),
        pltpu.VMEM((RPT,), jnp.float32),
        pltpu.SemaphoreType.DMA,
        pltpu.SemaphoreType.DMA,
    ],
)
def _sc_degree(eidx_hbm, zeros_hbm, degp_hbm, acc, eidx, onesv, stg, s0, s1):
    cid = lax.axis_index("c")
    sid = lax.axis_index("s")
    wid = cid * NS + sid

    pltpu.sync_copy(eidx_hbm.at[pl.ds(wid * NCHUNK, NCHUNK)], eidx)
    for j in range(CHUNK // 16):
        onesv[pl.ds(16 * j, 16)] = jnp.ones((16,), jnp.float32)
    pltpu.sync_copy(zeros_hbm, stg)
    pltpu.sync_copy(stg, acc.at[pl.ds(sid * RPT, RPT)])
    plsc.subcore_barrier()

    sems = (s0, s1)

    def fire(g, b):
        pltpu.async_copy(onesv, acc.at[eidx.at[g].at[1]], sems[b], add=True)

    def wait(g, b):
        pltpu.make_async_copy(onesv, acc.at[eidx.at[g].at[1]], sems[b]).wait()

    fire(0, 0)

    def body(i, carry):
        g = i * 2
        fire(g + 1, 1)
        wait(g, 0)

        @pl.when(g + 2 < NCHUNK)
        def _():
            fire(g + 2, 0)

        wait(g + 1, 1)
        return carry

    lax.fori_loop(0, NCHUNK // 2, body, 0)
    plsc.subcore_barrier()

    pltpu.sync_copy(acc.at[pl.ds(sid * RPT, RPT)], stg)
    pltpu.sync_copy(stg, degp_hbm.at[pl.ds(cid * NPAD + sid * RPT, RPT)])


@functools.partial(
    pl.kernel,
    out_type=jax.ShapeDtypeStruct((NC * NPAD, D), jnp.float32),
    mesh=_MESH,
    scratch_types=[
        pltpu.VMEM_SHARED((NPAD, D), jnp.float32),
        pltpu.VMEM((NIDX, 2, CHUNK), jnp.int32),
        pltpu.VMEM((NBUF, CHUNK, D), jnp.float32),
        pltpu.SemaphoreType.DMA,
        pltpu.SemaphoreType.DMA,
        pltpu.SemaphoreType.DMA,
        pltpu.SemaphoreType.DMA,
        pltpu.SemaphoreType.DMA,
        pltpu.SemaphoreType.DMA,
        pltpu.SemaphoreType.DMA,
        pltpu.SemaphoreType.DMA,
        pltpu.SemaphoreType.DMA,
        pltpu.SemaphoreType.DMA,
        pltpu.SemaphoreType.DMA,
        pltpu.SemaphoreType.DMA,
    ],
)
def _sc_aggregate(eidx_hbm, h2_hbm, accp_hbm, acc, eidx, rows,
                  i0, i1, i2, i3, i4, i5, i6, i7, r0, r1, r2, r3):
    cid = lax.axis_index("c")
    sid = lax.axis_index("s")
    wid = cid * NS + sid
    qbase = wid * NCHUNK

    def zfill(r, carry):
        for j in range(D // 16):
            rows[0, r, pl.ds(16 * j, 16)] = jnp.zeros((16,), jnp.float32)
        return carry

    lax.fori_loop(0, CHUNK, zfill, 0)
    for k in range(RPT // CHUNK):
        pltpu.sync_copy(rows.at[0],
                        acc.at[pl.ds(sid * RPT + k * CHUNK, CHUNK)])
    rem = RPT % CHUNK
    pltpu.sync_copy(rows.at[0].at[pl.ds(0, rem)],
                    acc.at[pl.ds(sid * RPT + (RPT // CHUNK) * CHUNK, rem)])
    plsc.subcore_barrier()

    isems = (i0, i1, i2, i3, i4, i5, i6, i7)
    rsems = (r0, r1, r2, r3)

    def fire_idx(g, s):
        pltpu.async_copy(eidx_hbm.at[qbase + g], eidx.at[s], isems[s])

    def fire_gather(g, s, b):
        pltpu.make_async_copy(eidx_hbm.at[qbase + g], eidx.at[s],
                              isems[s]).wait()
        pltpu.async_copy(h2_hbm.at[eidx.at[s].at[0]], rows.at[b], rsems[b])

    def drain(g, s, b):
        pltpu.make_async_copy(h2_hbm.at[eidx.at[s].at[0]], rows.at[b],
                              rsems[b]).wait()
        pltpu.sync_copy(rows.at[b], acc.at[eidx.at[s].at[1]], add=True)

    for j in range(NBUF + 2):
        fire_idx(j, j)
    for b in range(NBUF - 1):
        fire_gather(b, b, b)

    def body(i, carry):
        for u in range(NIDX):
            g = i * NIDX + u

            @pl.when(g + NBUF + 2 < NCHUNK)
            def _():
                fire_idx(g + NBUF + 2, (u + NBUF + 2) % NIDX)

            @pl.when(g + NBUF - 1 < NCHUNK)
            def _():
                fire_gather(g + NBUF - 1, (u + NBUF - 1) % NIDX,
                            (u + NBUF - 1) % NBUF)

            drain(g, u % NIDX, u % NBUF)
        return carry

    lax.fori_loop(0, NCHUNK // NIDX, body, 0)
    plsc.subcore_barrier()

    pltpu.sync_copy(
        acc.at[pl.ds(sid * RPT, RPT)],
        accp_hbm.at[pl.ds(cid * NPAD + sid * RPT, RPT)],
    )


def _dinv_diag(degp_ref, i):
    deg = degp_ref[pl.ds(i, 1), :] + degp_ref[pl.ds(NBLK + i, 1), :] + 1.0
    dinv = lax.rsqrt(deg)
    r = lax.broadcasted_iota(jnp.int32, (128, 128), 0)
    c = lax.broadcasted_iota(jnp.int32, (128, 128), 1)
    return jnp.where(r == c, dinv, 0.0)


def _tc_prescale_body(x_ref, w_ref, degp_ref, h2_ref):
    i = pl.program_id(0)
    h = jnp.dot(x_ref[...], w_ref[...], preferred_element_type=jnp.float32)
    h2_ref[...] = jnp.dot(_dinv_diag(degp_ref, i), h,
                          preferred_element_type=jnp.float32)


def _tc_combine_body(accp_ref, h2_ref, degp_ref, out_ref):
    i = pl.program_id(0)
    s = accp_ref[0] + accp_ref[1] + h2_ref[...]
    out_ref[...] = jnp.dot(_dinv_diag(degp_ref, i), s,
                           preferred_element_type=jnp.float32)


_tc_prescale = pl.pallas_call(
    _tc_prescale_body,
    grid=(NBLK,),
    in_specs=[
        pl.BlockSpec((128, D), lambda i: (i, 0)),
        pl.BlockSpec((D, D), lambda i: (0, 0)),
        pl.BlockSpec((2 * NBLK, 128), lambda i: (0, 0)),
    ],
    out_specs=pl.BlockSpec((128, D), lambda i: (i, 0)),
    out_shape=jax.ShapeDtypeStruct((NPAD, D), jnp.float32),
)

_tc_combine = pl.pallas_call(
    _tc_combine_body,
    grid=(NBLK,),
    in_specs=[
        pl.BlockSpec((NC, 128, D), lambda i: (0, i, 0)),
        pl.BlockSpec((128, D), lambda i: (i, 0)),
        pl.BlockSpec((2 * NBLK, 128), lambda i: (0, 0)),
    ],
    out_specs=pl.BlockSpec((128, D), lambda i: (i, 0)),
    out_shape=jax.ShapeDtypeStruct((N, D), jnp.float32),
)


def kernel(x, edge_index, W):
    src = edge_index[0].astype(jnp.int32)
    dst = edge_index[1].astype(jnp.int32)
    npadrows = NPAD - N
    pad = N + (lax.iota(jnp.int32, EPAD - E) % npadrows)
    srcp = jnp.concatenate([src, pad])
    dstp = jnp.concatenate([dst, pad])
    eidx = jnp.stack(
        [srcp.reshape(NW * NCHUNK, CHUNK), dstp.reshape(NW * NCHUNK, CHUNK)],
        axis=1,
    )
    xp = jnp.zeros((NPAD, D), jnp.float32).at[:N].set(x)

    za = jnp.zeros((RPT,), jnp.float32)

    degp = _sc_degree(eidx, za).reshape(2 * NBLK, 128)
    h2 = _tc_prescale(xp, W, degp)
    accp = _sc_aggregate(eidx, h2).reshape(NC, NPAD, D)
    return _tc_combine(accp, h2, degp)


# speedup vs baseline: 33.1401x; 1.0081x over previous
import functools

import jax
import jax.numpy as jnp
from jax import lax
from jax.experimental import pallas as pl
from jax.experimental.pallas import tpu as pltpu
from jax.experimental.pallas import tpu_sc as plsc

N = 10000
E = 320000
D = 128

NC = 2
NS = 16
NW = NC * NS

NBLK = 79
NPAD = NBLK * 128
CHUNK = 80
NCHUNK = 128
EPT = NCHUNK * CHUNK
EPAD = NW * EPT
RPT = NPAD // NS
NBUF = 4
NIDX = 8

_MESH = plsc.VectorSubcoreMesh(core_axis_name="c", subcore_axis_name="s")


@functools.partial(
    pl.kernel,
    out_type=jax.ShapeDtypeStruct((NC * NPAD,), jnp.float32),
    mesh=_MESH,
    scratch_types=[
        pltpu.VMEM_SHARED((NPAD,), jnp.float32),
        pltpu.VMEM((NCHUNK, 2, CHUNK), jnp.int32),
        pltpu.VMEM((CHUNK,), jnp.float32),
        pltpu.VMEM((RPT,), jnp.float32),
        pltpu.SemaphoreType.DMA,
        pltpu.SemaphoreType.DMA,
    ],
)
def _sc_degree(eidx_hbm, zeros_hbm, degp_hbm, acc, eidx, onesv, stg, s0, s1):
    cid = lax.axis_index("c")
    sid = lax.axis_index("s")
    wid = cid * NS + sid

    pltpu.sync_copy(eidx_hbm.at[pl.ds(wid * NCHUNK, NCHUNK)], eidx)
    for j in range(CHUNK // 16):
        onesv[pl.ds(16 * j, 16)] = jnp.ones((16,), jnp.float32)
    pltpu.sync_copy(zeros_hbm, stg)
    pltpu.sync_copy(stg, acc.at[pl.ds(sid * RPT, RPT)])
    plsc.subcore_barrier()

    sems = (s0, s1)

    def fire(g, b):
        pltpu.async_copy(onesv, acc.at[eidx.at[g].at[1]], sems[b], add=True)

    def wait(g, b):
        pltpu.make_async_copy(onesv, acc.at[eidx.at[g].at[1]], sems[b]).wait()

    fire(0, 0)

    def body(i, carry):
        g = i * 2
        fire(g + 1, 1)
        wait(g, 0)

        @pl.when(g + 2 < NCHUNK)
        def _():
            fire(g + 2, 0)

        wait(g + 1, 1)
        return carry

    lax.fori_loop(0, NCHUNK // 2, body, 0)
    plsc.subcore_barrier()

    pltpu.sync_copy(acc.at[pl.ds(sid * RPT, RPT)], stg)
    pltpu.sync_copy(stg, degp_hbm.at[pl.ds(cid * NPAD + sid * RPT, RPT)])


@functools.partial(
    pl.kernel,
    out_type=jax.ShapeDtypeStruct((NC * NPAD, D), jnp.float32),
    mesh=_MESH,
    scratch_types=[
        pltpu.VMEM_SHARED((NPAD, D), jnp.float32),
        pltpu.VMEM((NIDX, 2, CHUNK), jnp.int32),
        pltpu.VMEM((NBUF, CHUNK, D), jnp.float32),
        pltpu.SemaphoreType.DMA,
        pltpu.SemaphoreType.DMA,
        pltpu.SemaphoreType.DMA,
        pltpu.SemaphoreType.DMA,
        pltpu.SemaphoreType.DMA,
        pltpu.SemaphoreType.DMA,
        pltpu.SemaphoreType.DMA,
        pltpu.SemaphoreType.DMA,
        pltpu.SemaphoreType.DMA,
        pltpu.SemaphoreType.DMA,
        pltpu.SemaphoreType.DMA,
        pltpu.SemaphoreType.DMA,
        pltpu.SemaphoreType.DMA,
        pltpu.SemaphoreType.DMA,
        pltpu.SemaphoreType.DMA,
        pltpu.SemaphoreType.DMA,
    ],
)
def _sc_aggregate(eidx_hbm, h2_hbm, accp_hbm, acc, eidx, rows,
                  i0, i1, i2, i3, i4, i5, i6, i7, r0, r1, r2, r3,
                  w0, w1, w2, w3):
    cid = lax.axis_index("c")
    sid = lax.axis_index("s")
    wid = cid * NS + sid
    qbase = wid * NCHUNK

    def zfill(r, carry):
        for j in range(D // 16):
            rows[0, r, pl.ds(16 * j, 16)] = jnp.zeros((16,), jnp.float32)
        return carry

    lax.fori_loop(0, CHUNK, zfill, 0)
    for k in range(RPT // CHUNK):
        pltpu.sync_copy(rows.at[0],
                        acc.at[pl.ds(sid * RPT + k * CHUNK, CHUNK)])
    rem = RPT % CHUNK
    pltpu.sync_copy(rows.at[0].at[pl.ds(0, rem)],
                    acc.at[pl.ds(sid * RPT + (RPT // CHUNK) * CHUNK, rem)])
    plsc.subcore_barrier()

    isems = (i0, i1, i2, i3, i4, i5, i6, i7)
    rsems = (r0, r1, r2, r3)
    wsems = (w0, w1, w2, w3)

    def fire_idx(g, s):
        pltpu.async_copy(eidx_hbm.at[qbase + g], eidx.at[s], isems[s])

    def fire_gather(g, s, b):
        pltpu.make_async_copy(eidx_hbm.at[qbase + g], eidx.at[s],
                              isems[s]).wait()
        pltpu.async_copy(h2_hbm.at[eidx.at[s].at[0]], rows.at[b], rsems[b])

    def wait_scatter(g, s, b):
        pltpu.make_async_copy(rows.at[b], acc.at[eidx.at[s].at[1]],
                              wsems[b]).wait()

    def drain(g, s, b):
        pltpu.make_async_copy(h2_hbm.at[eidx.at[s].at[0]], rows.at[b],
                              rsems[b]).wait()
        pltpu.async_copy(rows.at[b], acc.at[eidx.at[s].at[1]], wsems[b],
                         add=True)

    for j in range(NBUF + 2):
        fire_idx(j, j)
    for b in range(NBUF - 1):
        fire_gather(b, b, b)

    def body(i, carry):
        for u in range(NIDX):
            g = i * NIDX + u

            @pl.when(g + NBUF + 2 < NCHUNK)
            def _():
                fire_idx(g + NBUF + 2, (u + NBUF + 2) % NIDX)

            @pl.when(g + NBUF - 1 < NCHUNK)
            def _():
                @pl.when(g >= 1)
                def _():
                    wait_scatter(g - 1, (u + NBUF - 1) % NIDX,
                                 (u + NBUF - 1) % NBUF)

                fire_gather(g + NBUF - 1, (u + NBUF - 1) % NIDX,
                            (u + NBUF - 1) % NBUF)

            drain(g, u % NIDX, u % NBUF)
        return carry

    lax.fori_loop(0, NCHUNK // NIDX, body, 0)
    for b in range(NBUF):
        wait_scatter(NCHUNK - NBUF + b, (NCHUNK - NBUF + b) % NIDX, b)
    plsc.subcore_barrier()

    pltpu.sync_copy(
        acc.at[pl.ds(sid * RPT, RPT)],
        accp_hbm.at[pl.ds(cid * NPAD + sid * RPT, RPT)],
    )


def _dinv_diag(degp_ref, i):
    deg = degp_ref[pl.ds(i, 1), :] + degp_ref[pl.ds(NBLK + i, 1), :] + 1.0
    dinv = lax.rsqrt(deg)
    r = lax.broadcasted_iota(jnp.int32, (128, 128), 0)
    c = lax.broadcasted_iota(jnp.int32, (128, 128), 1)
    return jnp.where(r == c, dinv, 0.0)


def _tc_prescale_body(x_ref, w_ref, degp_ref, h2_ref):
    i = pl.program_id(0)
    rowid = lax.broadcasted_iota(jnp.int32, (128, D), 0) + i * 128
    xv = jnp.where(rowid < N, x_ref[...], 0.0)
    h = jnp.dot(xv, w_ref[...], preferred_element_type=jnp.float32)
    h2_ref[...] = jnp.dot(_dinv_diag(degp_ref, i), h,
                          preferred_element_type=jnp.float32)


def _tc_combine_body(accp_ref, h2_ref, degp_ref, out_ref):
    i = pl.program_id(0)
    s = accp_ref[0] + accp_ref[1] + h2_ref[...]
    out_ref[...] = jnp.dot(_dinv_diag(degp_ref, i), s,
                           preferred_element_type=jnp.float32)


_tc_prescale = pl.pallas_call(
    _tc_prescale_body,
    grid=(NBLK,),
    in_specs=[
        pl.BlockSpec((128, D), lambda i: (i, 0)),
        pl.BlockSpec((D, D), lambda i: (0, 0)),
        pl.BlockSpec((2 * NBLK, 128), lambda i: (0, 0)),
    ],
    out_specs=pl.BlockSpec((128, D), lambda i: (i, 0)),
    out_shape=jax.ShapeDtypeStruct((NPAD, D), jnp.float32),
)

_tc_combine = pl.pallas_call(
    _tc_combine_body,
    grid=(NBLK,),
    in_specs=[
        pl.BlockSpec((NC, 128, D), lambda i: (0, i, 0)),
        pl.BlockSpec((128, D), lambda i: (i, 0)),
        pl.BlockSpec((2 * NBLK, 128), lambda i: (0, 0)),
    ],
    out_specs=pl.BlockSpec((128, D), lambda i: (i, 0)),
    out_shape=jax.ShapeDtypeStruct((N, D), jnp.float32),
)


def kernel(x, edge_index, W):
    src = edge_index[0].astype(jnp.int32)
    dst = edge_index[1].astype(jnp.int32)
    npadrows = NPAD - N
    pad = N + (lax.iota(jnp.int32, EPAD - E) % npadrows)
    srcp = jnp.concatenate([src, pad])
    dstp = jnp.concatenate([dst, pad])
    eidx = jnp.stack(
        [srcp.reshape(NW * NCHUNK, CHUNK), dstp.reshape(NW * NCHUNK, CHUNK)],
        axis=1,
    )
    za = jnp.zeros((RPT,), jnp.float32)

    degp = _sc_degree(eidx, za).reshape(2 * NBLK, 128)
    h2 = _tc_prescale(x, W, degp)
    accp = _sc_aggregate(eidx, h2).reshape(NC, NPAD, D)
    return _tc_combine(accp, h2, degp)


# speedup vs baseline: 33.1473x; 1.0002x over previous
import functools

import jax
import jax.numpy as jnp
from jax import lax
from jax.experimental import pallas as pl
from jax.experimental.pallas import tpu as pltpu
from jax.experimental.pallas import tpu_sc as plsc

N = 10000
E = 320000
D = 128

NC = 2
NS = 16
NW = NC * NS

NBLK = 79
NPAD = NBLK * 128
CHUNK = 80
NCHUNK = 128
EPT = NCHUNK * CHUNK
EPAD = NW * EPT
RPT = NPAD // NS
NBUF = 4
NIDX = 8

_MESH = plsc.VectorSubcoreMesh(
    core_axis_name="c", subcore_axis_name="s", num_cores=NC, num_subcores=NS
)


@functools.partial(
    pl.kernel,
    out_type=jax.ShapeDtypeStruct((NC * NPAD,), jnp.float32),
    mesh=_MESH,
    scratch_types=[
        pltpu.VMEM_SHARED((NPAD,), jnp.float32),
        pltpu.VMEM((NCHUNK, 2, CHUNK), jnp.int32),
        pltpu.VMEM((CHUNK,), jnp.float32),
        pltpu.VMEM((RPT,), jnp.float32),
        pltpu.SemaphoreType.DMA,
        pltpu.SemaphoreType.DMA,
    ],
)
def _sc_degree(eidx_hbm, zeros_hbm, degp_hbm, acc, eidx, onesv, stg, s0, s1):
    cid = lax.axis_index("c")
    sid = lax.axis_index("s")
    wid = cid * NS + sid

    pltpu.sync_copy(eidx_hbm.at[pl.ds(wid * NCHUNK, NCHUNK)], eidx)
    for j in range(CHUNK // 16):
        onesv[pl.ds(16 * j, 16)] = jnp.ones((16,), jnp.float32)
    pltpu.sync_copy(zeros_hbm, stg)
    pltpu.sync_copy(stg, acc.at[pl.ds(sid * RPT, RPT)])
    plsc.subcore_barrier()

    sems = (s0, s1)

    def fire(g, b):
        pltpu.async_copy(onesv, acc.at[eidx.at[g].at[1]], sems[b], add=True)

    def wait(g, b):
        pltpu.make_async_copy(onesv, acc.at[eidx.at[g].at[1]], sems[b]).wait()

    fire(0, 0)

    def body(i, carry):
        g = i * 2
        fire(g + 1, 1)
        wait(g, 0)

        @pl.when(g + 2 < NCHUNK)
        def _():
            fire(g + 2, 0)

        wait(g + 1, 1)
        return carry

    lax.fori_loop(0, NCHUNK // 2, body, 0)
    plsc.subcore_barrier()

    pltpu.sync_copy(acc.at[pl.ds(sid * RPT, RPT)], stg)
    pltpu.sync_copy(stg, degp_hbm.at[pl.ds(cid * NPAD + sid * RPT, RPT)])


@functools.partial(
    pl.kernel,
    out_type=jax.ShapeDtypeStruct((NC * NPAD, D), jnp.float32),
    mesh=_MESH,
    scratch_types=[
        pltpu.VMEM_SHARED((NPAD, D), jnp.float32),
        pltpu.VMEM((NIDX, 2, CHUNK), jnp.int32),
        pltpu.VMEM((NBUF, CHUNK, D), jnp.float32),
        pltpu.SemaphoreType.DMA,
        pltpu.SemaphoreType.DMA,
        pltpu.SemaphoreType.DMA,
        pltpu.SemaphoreType.DMA,
        pltpu.SemaphoreType.DMA,
        pltpu.SemaphoreType.DMA,
        pltpu.SemaphoreType.DMA,
        pltpu.SemaphoreType.DMA,
        pltpu.SemaphoreType.DMA,
        pltpu.SemaphoreType.DMA,
        pltpu.SemaphoreType.DMA,
        pltpu.SemaphoreType.DMA,
        pltpu.SemaphoreType.DMA,
        pltpu.SemaphoreType.DMA,
        pltpu.SemaphoreType.DMA,
        pltpu.SemaphoreType.DMA,
    ],
)
def _sc_aggregate(eidx_hbm, h2_hbm, accp_hbm, acc, eidx, rows,
                  i0, i1, i2, i3, i4, i5, i6, i7, r0, r1, r2, r3,
                  w0, w1, w2, w3):
    cid = lax.axis_index("c")
    sid = lax.axis_index("s")
    wid = cid * NS + sid
    qbase = wid * NCHUNK

    def zfill(r, carry):
        for j in range(D // 16):
            rows[0, r, pl.ds(16 * j, 16)] = jnp.zeros((16,), jnp.float32)
        return carry

    lax.fori_loop(0, CHUNK, zfill, 0)
    for k in range(RPT // CHUNK):
        pltpu.sync_copy(rows.at[0],
                        acc.at[pl.ds(sid * RPT + k * CHUNK, CHUNK)])
    rem = RPT % CHUNK
    pltpu.sync_copy(rows.at[0].at[pl.ds(0, rem)],
                    acc.at[pl.ds(sid * RPT + (RPT // CHUNK) * CHUNK, rem)])
    plsc.subcore_barrier()

    isems = (i0, i1, i2, i3, i4, i5, i6, i7)
    rsems = (r0, r1, r2, r3)
    wsems = (w0, w1, w2, w3)

    def fire_idx(g, s):
        pltpu.async_copy(eidx_hbm.at[qbase + g], eidx.at[s], isems[s])

    def fire_gather(g, s, b):
        pltpu.make_async_copy(eidx_hbm.at[qbase + g], eidx.at[s],
                              isems[s]).wait()
        pltpu.async_copy(h2_hbm.at[eidx.at[s].at[0]], rows.at[b], rsems[b])

    def wait_scatter(g, s, b):
        pltpu.make_async_copy(rows.at[b], acc.at[eidx.at[s].at[1]],
                              wsems[b]).wait()

    def drain(g, s, b):
        pltpu.make_async_copy(h2_hbm.at[eidx.at[s].at[0]], rows.at[b],
                              rsems[b]).wait()
        pltpu.async_copy(rows.at[b], acc.at[eidx.at[s].at[1]], wsems[b],
                         add=True)

    for j in range(NBUF + 2):
        fire_idx(j, j)
    for b in range(NBUF - 1):
        fire_gather(b, b, b)

    def body(i, carry):
        for u in range(NIDX):
            g = i * NIDX + u

            @pl.when(g + NBUF + 2 < NCHUNK)
            def _():
                fire_idx(g + NBUF + 2, (u + NBUF + 2) % NIDX)

            @pl.when(g + NBUF - 1 < NCHUNK)
            def _():
                @pl.when(g >= 1)
                def _():
                    wait_scatter(g - 1, (u + NBUF - 1) % NIDX,
                                 (u + NBUF - 1) % NBUF)

                fire_gather(g + NBUF - 1, (u + NBUF - 1) % NIDX,
                            (u + NBUF - 1) % NBUF)

            drain(g, u % NIDX, u % NBUF)
        return carry

    lax.fori_loop(0, NCHUNK // NIDX, body, 0)
    for b in range(NBUF):
        wait_scatter(NCHUNK - NBUF + b, (NCHUNK - NBUF + b) % NIDX, b)
    plsc.subcore_barrier()

    pltpu.sync_copy(
        acc.at[pl.ds(sid * RPT, RPT)],
        accp_hbm.at[pl.ds(cid * NPAD + sid * RPT, RPT)],
    )


def _dinv_diag(degp_ref, i):
    deg = degp_ref[pl.ds(i, 1), :] + degp_ref[pl.ds(NBLK + i, 1), :] + 1.0
    dinv = lax.rsqrt(deg)
    r = lax.broadcasted_iota(jnp.int32, (128, 128), 0)
    c = lax.broadcasted_iota(jnp.int32, (128, 128), 1)
    return jnp.where(r == c, dinv, 0.0)


def _tc_prescale_body(x_ref, w_ref, degp_ref, h2_ref):
    i = pl.program_id(0)
    rowid = lax.broadcasted_iota(jnp.int32, (128, D), 0) + i * 128
    xv = jnp.where(rowid < N, x_ref[...], 0.0)
    h = jnp.dot(xv, w_ref[...], preferred_element_type=jnp.float32)
    h2_ref[...] = jnp.dot(_dinv_diag(degp_ref, i), h,
                          preferred_element_type=jnp.float32)


def _tc_combine_body(accp_ref, h2_ref, degp_ref, out_ref):
    i = pl.program_id(0)
    s = accp_ref[0] + accp_ref[1] + h2_ref[...]
    out_ref[...] = jnp.dot(_dinv_diag(degp_ref, i), s,
                           preferred_element_type=jnp.float32)


_tc_prescale = pl.pallas_call(
    _tc_prescale_body,
    grid=(NBLK,),
    in_specs=[
        pl.BlockSpec((128, D), lambda i: (i, 0)),
        pl.BlockSpec((D, D), lambda i: (0, 0)),
        pl.BlockSpec((2 * NBLK, 128), lambda i: (0, 0)),
    ],
    out_specs=pl.BlockSpec((128, D), lambda i: (i, 0)),
    out_shape=jax.ShapeDtypeStruct((NPAD, D), jnp.float32),
)

_tc_combine = pl.pallas_call(
    _tc_combine_body,
    grid=(NBLK,),
    in_specs=[
        pl.BlockSpec((NC, 128, D), lambda i: (0, i, 0)),
        pl.BlockSpec((128, D), lambda i: (i, 0)),
        pl.BlockSpec((2 * NBLK, 128), lambda i: (0, 0)),
    ],
    out_specs=pl.BlockSpec((128, D), lambda i: (i, 0)),
    out_shape=jax.ShapeDtypeStruct((N, D), jnp.float32),
)


def kernel(x, edge_index, W):
    src = edge_index[0].astype(jnp.int32)
    dst = edge_index[1].astype(jnp.int32)
    npadrows = NPAD - N
    pad = N + (lax.iota(jnp.int32, EPAD - E) % npadrows)
    srcp = jnp.concatenate([src, pad])
    dstp = jnp.concatenate([dst, pad])
    eidx = jnp.stack(
        [srcp.reshape(NW * NCHUNK, CHUNK), dstp.reshape(NW * NCHUNK, CHUNK)],
        axis=1,
    )
    za = jnp.zeros((RPT,), jnp.float32)

    degp = _sc_degree(eidx, za).reshape(2 * NBLK, 128)
    h2 = _tc_prescale(x, W, degp)
    accp = _sc_aggregate(eidx, h2).reshape(NC, NPAD, D)
    return _tc_combine(accp, h2, degp)
